# baseline (speedup 1.0000x reference)
import functools

import jax
import jax.numpy as jnp
from jax import lax
from jax.experimental import pallas as pl
from jax.experimental.pallas import tpu as pltpu
from jax.experimental.pallas import tpu_sc as plsc

NUM_NODES = 1000000
OUT_DIMS = 32
BATCH = 16384
HIST = 200

LANES = 128
NBT = BATCH // LANES
NC = 2
NS = 16
NW = NC * NS
BT_PER_W = NBT // NW
STEPS = BT_PER_W * HIST
HBLK = 40
PIPE = 4


@functools.partial(
    pl.kernel,
    mesh=plsc.VectorSubcoreMesh(core_axis_name="c", subcore_axis_name="s"),
    compiler_params=pltpu.CompilerParams(
        use_tc_tiling_on_sc=False, needs_layout_passes=False
    ),
    out_type=jax.ShapeDtypeStruct((HIST, 4, NBT, 8, LANES), jnp.float32),
    scratch_types=[
        pltpu.VMEM((2, HBLK, LANES), jnp.int32),
        pltpu.VMEM((PIPE, LANES, OUT_DIMS), jnp.float32),
        pltpu.VMEM((2, 4, 8, LANES), jnp.float32),
        pltpu.SemaphoreType.DMA,
        pltpu.SemaphoreType.DMA,
    ],
)
def _emb_lookup(idx_hbm, table_hbm, out_hbm, idx_v, rows_v, tile_v, gsem, wsem):
    wid = lax.axis_index("s") * NC + lax.axis_index("c")
    col0 = wid * BT_PER_W * LANES

    lane16 = lax.iota(jnp.int32, 16)
    bl_vecs = [lane16 + j * 16 for j in range(8)]

    def stage_idx(g):
        h = g % HIST
        bt = g // HIST
        pltpu.sync_copy(
            idx_hbm.at[pl.ds(h, HBLK), pl.ds(col0 + bt * LANES, LANES)],
            idx_v.at[(g // HBLK) % 2],
        )

    def gather(g, wait):
        cp = (
            pltpu.make_async_copy
            if wait
            else lambda s, d, m: pltpu.async_copy(s, d, m)
        )
        c = cp(
            table_hbm.at[idx_v.at[(g // HBLK) % 2, g % HBLK]],
            rows_v.at[g % PIPE],
            gsem,
        )
        if wait:
            c.wait()

    def out_slice(g):
        return out_hbm.at[g % HIST, pl.ds(0, 4), wid * BT_PER_W + g // HIST]

    stage_idx(0)
    for gi in range(PIPE - 1):
        gather(gi, False)

    def body(g, carry):
        g3 = g + PIPE - 1
        st = g % 2

        @pl.when(jnp.logical_and(g3 < STEPS, g3 % HBLK == 0))
        def _stage():
            stage_idx(g3)

        @pl.when(g3 < STEPS)
        def _fire():
            gather(g3, False)

        gather(g, True)

        @pl.when(g >= 2)
        def _reclaim():
            pltpu.make_async_copy(tile_v.at[st], out_slice(g), wsem).wait()

        rows = rows_v.at[g % PIPE]
        for d in range(OUT_DIMS):
            dcol = jnp.full((16,), d, jnp.int32)
            for j in range(8):
                tile_v[st, d // 8, d % 8, pl.ds(j * 16, 16)] = plsc.load_gather(
                    rows, [bl_vecs[j], dcol]
                )

        pltpu.async_copy(tile_v.at[st], out_slice(g), wsem)
        return carry

    lax.fori_loop(0, STEPS, body, 0)
    pltpu.make_async_copy(tile_v.at[0], out_slice(STEPS - 2), wsem).wait()
    pltpu.make_async_copy(tile_v.at[1], out_slice(STEPS - 1), wsem).wait()


def kernel(idx, table):
    out5 = _emb_lookup(idx.T.astype(jnp.int32), table)
    return jnp.transpose(out5, (2, 4, 0, 1, 3)).reshape(BATCH, HIST, OUT_DIMS)


# speedup vs baseline: 1.0885x; 1.0885x over previous
import functools

import jax
import jax.numpy as jnp
from jax import lax
from jax.experimental import pallas as pl
from jax.experimental.pallas import tpu as pltpu
from jax.experimental.pallas import tpu_sc as plsc

NUM_NODES = 1000000
OUT_DIMS = 32
BATCH = 16384
HIST = 200

LANES = 128
NBT = BATCH // LANES
NC = 2
NS = 16
NW = NC * NS
BT_PER_W = NBT // NW
STEPS = BT_PER_W * HIST
HBLK = 40
PIPE = 4


@functools.partial(
    pl.kernel,
    mesh=plsc.VectorSubcoreMesh(core_axis_name="c", subcore_axis_name="s"),
    compiler_params=pltpu.CompilerParams(
        use_tc_tiling_on_sc=False, needs_layout_passes=False
    ),
    out_type=jax.ShapeDtypeStruct((HIST, 4, NBT, 8, LANES), jnp.float32),
    scratch_types=[
        pltpu.VMEM((2, HBLK, LANES), jnp.int32),
        pltpu.VMEM((PIPE, LANES, OUT_DIMS), jnp.float32),
        pltpu.VMEM((2, 4, 8, LANES), jnp.float32),
        pltpu.SemaphoreType.DMA,
        pltpu.SemaphoreType.DMA,
    ],
)
def _emb_lookup(idx_hbm, table_hbm, out_hbm, idx_v, rows_v, tile_v, gsem, wsem):
    wid = lax.axis_index("s") * NC + lax.axis_index("c")
    col0 = wid * BT_PER_W * LANES

    lane16 = lax.iota(jnp.int32, 16)
    bl_vecs = [lane16 + j * 16 for j in range(8)]

    def stage_idx(g):
        h = g % HIST
        bt = g // HIST
        pltpu.sync_copy(
            idx_hbm.at[pl.ds(h, HBLK), pl.ds(col0 + bt * LANES, LANES)],
            idx_v.at[(g // HBLK) % 2],
        )

    def gather(g, wait):
        cp = (
            pltpu.make_async_copy
            if wait
            else lambda s, d, m: pltpu.async_copy(s, d, m)
        )
        c = cp(
            table_hbm.at[idx_v.at[(g // HBLK) % 2, g % HBLK]],
            rows_v.at[g % PIPE],
            gsem,
        )
        if wait:
            c.wait()

    def out_slice(g):
        return out_hbm.at[g % HIST, pl.ds(0, 4), wid * BT_PER_W + g // HIST]

    stage_idx(0)
    for gi in range(PIPE - 1):
        gather(gi, False)

    def body(g, carry):
        g3 = g + PIPE - 1
        st = g % 2

        @pl.when(jnp.logical_and(g3 < STEPS, g3 % HBLK == 0))
        def _stage():
            stage_idx(g3)

        @pl.when(g3 < STEPS)
        def _fire():
            gather(g3, False)

        gather(g, True)

        @pl.when(g >= 2)
        def _reclaim():
            pltpu.make_async_copy(tile_v.at[st], out_slice(g), wsem).wait()

        rows = rows_v.at[g % PIPE]

        @plsc.parallel_loop(0, OUT_DIMS, unroll=8)
        def _transpose(d):
            dcol = jnp.full((16,), 0, jnp.int32) + d
            vals = [plsc.load_gather(rows, [bl_vecs[j], dcol]) for j in range(8)]
            for j in range(8):
                tile_v[st, d // 8, d % 8, pl.ds(j * 16, 16)] = vals[j]

        pltpu.async_copy(tile_v.at[st], out_slice(g), wsem)
        return carry

    lax.fori_loop(0, STEPS, body, 0)
    pltpu.make_async_copy(tile_v.at[0], out_slice(STEPS - 2), wsem).wait()
    pltpu.make_async_copy(tile_v.at[1], out_slice(STEPS - 1), wsem).wait()


def kernel(idx, table):
    out5 = _emb_lookup(idx.T.astype(jnp.int32), table)
    return jnp.transpose(out5, (2, 4, 0, 1, 3)).reshape(BATCH, HIST, OUT_DIMS)
